# baseline (speedup 1.0000x reference)
import functools

import jax
import jax.numpy as jnp
from jax import lax
from jax.experimental import pallas as pl
from jax.experimental.pallas import tpu as pltpu
from jax.experimental.pallas import tpu_sc as plsc

N = 10000
E = 320000
D = 128
H = 16
FM = 16
OUT = 16

NC = 2
NS = 16
NW = NC * NS
EPW = E // NW
G = 80
CH_G = 25
CH = CH_G * G
NCH = EPW // CH
VG = G // 16

NP = N // 8
NSG = N // 16
SG_T = NSG // NS

_SPLAT_DNUMS = lax.GatherDimensionNumbers(
    offset_dims=(), collapsed_slice_dims=(0,), start_index_map=(0,))


def _splat_lane(v, j):
    idx = jnp.full((16, 1), j, jnp.int32)
    return lax.gather(v, idx, _SPLAT_DNUMS, (1,),
                      mode=lax.GatherScatterMode.PROMISE_IN_BOUNDS)


def _pre_body(xp_ref, bdw0_ref, kv_ref, h_ref, f12_ref):
    hb_p = jnp.dot(xp_ref[...], bdw0_ref[...],
                   preferred_element_type=jnp.float32)
    h_ref[...] = hb_p
    f12_ref[...] = jnp.dot(hb_p, kv_ref[...],
                           preferred_element_type=jnp.float32)


def _pre(x_p, bdw0, kv):
    return pl.pallas_call(
        _pre_body,
        out_shape=[
            jax.ShapeDtypeStruct((NP, 128), jnp.float32),
            jax.ShapeDtypeStruct((NP, 16), jnp.float32),
        ],
    )(x_p, bdw0, kv)


def _sc_edges(ei, f12i, h, zu, zs):
    mesh = plsc.VectorSubcoreMesh(core_axis_name="c", subcore_axis_name="s")

    @functools.partial(
        pl.kernel,
        mesh=mesh,
        out_type=[
            jax.ShapeDtypeStruct((NC, N, H), jnp.float32),
            jax.ShapeDtypeStruct((NC, N, 16), jnp.float32),
        ],
        scratch_types=[
            pltpu.VMEM((2 * N,), jnp.float32),
            pltpu.VMEM((EPW,), jnp.int32),
            pltpu.VMEM((EPW,), jnp.int32),
            pltpu.VMEM((CH,), jnp.float32),
            pltpu.VMEM((CH,), jnp.float32),
            pltpu.VMEM((CH, H), jnp.float32),
            pltpu.VMEM((CH, H), jnp.float32),
            pltpu.VMEM(((SG_T + 1) * 16,), jnp.float32),
            pltpu.VMEM_SHARED((N, H), jnp.float32),
            pltpu.VMEM_SHARED((N,), jnp.float32),
            pltpu.SemaphoreType.DMA,
            pltpu.SemaphoreType.DMA,
            pltpu.SemaphoreType.DMA,
            pltpu.SemaphoreType.DMA,
            pltpu.SemaphoreType.DMA,
        ],
        compiler_params=pltpu.CompilerParams(
            needs_layout_passes=False, use_tc_tiling_on_sc=False),
    )
    def k(ei_hbm, f12_hbm, h_hbm, zu_hbm, zs_hbm,
          u_out, sw_out, f12_v, row_f, col_f, val0, val1, hr0, hr1,
          s_loc, u_sh, s_sh, sem_i, sem_g0, sem_g1, sem_s0, sem_s1):
        cid = lax.axis_index("c")
        sid = lax.axis_index("s")
        wid = cid * NS + sid
        base_e = wid * EPW

        vals = (val0, val1)
        hrows = (hr0, hr1)
        sem_g = (sem_g0, sem_g1)
        sem_s = (sem_s0, sem_s1)

        cp_r = pltpu.async_copy(ei_hbm.at[0, pl.ds(base_e, EPW)], row_f, sem_i)
        cp_c = pltpu.async_copy(ei_hbm.at[1, pl.ds(base_e, EPW)], col_f, sem_i)

        @pl.when(sid == 0)
        def _():
            pltpu.sync_copy(zu_hbm, u_sh)
            pltpu.sync_copy(zs_hbm, s_sh)

        pltpu.sync_copy(f12_hbm, f12_v)
        cp_r.wait()
        cp_c.wait()

        plsc.subcore_barrier()

        def fire_gathers(k_ch, b):
            def body(g, _):
                pltpu.async_copy(
                    h_hbm.at[col_f.at[pl.ds(k_ch * CH + g * G, G)]],
                    hrows[b].at[pl.ds(g * G, G)],
                    sem_g[b])
                return 0
            lax.fori_loop(0, CH_G, body, 0)

        def drain_gathers(b):
            pltpu.make_async_copy(h_hbm.at[pl.ds(0, CH)], hrows[b],
                                  sem_g[b]).wait()

        def drain_scatters(b):
            pltpu.make_async_copy(h_hbm.at[pl.ds(0, CH)], hrows[b],
                                  sem_s[b]).wait()
            pltpu.make_async_copy(f12_hbm.at[pl.ds(0, CH)], vals[b],
                                  sem_s[b]).wait()

        def compute_and_scatter(k_ch, b):
            def val_body(q, _):
                for tt in range(VG):
                    loc = q * G + tt * 16
                    e0 = k_ch * CH + loc
                    rv = row_f[pl.ds(e0, 16)]
                    cv = col_f[pl.ds(e0, 16)]
                    f1g = plsc.load_gather(f12_v, [rv + rv])
                    f2g = plsc.load_gather(f12_v, [cv + cv + 1])
                    logit = f1g + f2g
                    sg = 1.0 / (1.0 + jnp.exp(-logit))
                    vals[b][pl.ds(loc, 16)] = jnp.exp(sg)
                return 0
            lax.fori_loop(0, CH_G, val_body, 0)

            def group_body(g, _):
                for t in range(VG):
                    loc = g * G + t * 16
                    val = vals[b][pl.ds(loc, 16)]
                    for j in range(16):
                        splat = _splat_lane(val, j)
                        hrows[b][loc + j] = hrows[b][loc + j] * splat
                idx_slice = row_f.at[pl.ds(k_ch * CH + g * G, G)]
                pltpu.async_copy(hrows[b].at[pl.ds(g * G, G)],
                                 u_sh.at[idx_slice], sem_s[b], add=True)
                pltpu.async_copy(vals[b].at[pl.ds(g * G, G)],
                                 s_sh.at[idx_slice], sem_s[b], add=True)
                return 0
            lax.fori_loop(0, CH_G, group_body, 0)

        fire_gathers(0, 0)
        for k_ch in range(NCH):
            b = k_ch % 2
            nb = 1 - b
            if k_ch >= 1:
                drain_scatters(nb)
            if k_ch + 1 < NCH:
                fire_gathers(k_ch + 1, nb)
            drain_gathers(b)
            compute_and_scatter(k_ch, b)
        drain_scatters((NCH - 1) % 2)

        plsc.subcore_barrier()

        @pl.when(sid == 0)
        def _():
            pltpu.sync_copy(u_sh, u_out.at[cid])

        NPT = SG_T * 16
        pltpu.sync_copy(s_sh.at[pl.ds(sid * NPT, NPT)],
                        s_loc.at[pl.ds(0, NPT)])

        @pl.when(sid == NS - 1)
        def _():
            pltpu.sync_copy(s_sh.at[pl.ds(NS * NPT, 16)],
                            s_loc.at[pl.ds(NPT, 16)])

        def sw_body(i, _):
            s16 = s_loc[pl.ds(i * 16, 16)]
            for j in range(16):
                hr0[i * 16 + j] = _splat_lane(s16, j)
            return 0
        lax.fori_loop(0, SG_T, sw_body, 0)
        pltpu.sync_copy(hr0.at[pl.ds(0, NPT)],
                        sw_out.at[cid, pl.ds(sid * NPT, NPT)])

        @pl.when(sid == NS - 1)
        def _():
            s16 = s_loc[pl.ds(NPT, 16)]
            for j in range(16):
                hr1[j] = _splat_lane(s16, j)
            pltpu.sync_copy(hr1.at[pl.ds(0, 16)],
                            sw_out.at[cid, pl.ds(NS * NPT, 16)])

    return k(ei, f12i, h, zu, zs)


def _post_body(xp_ref, bdf_ref, bdf2_ref, u_ref, sw_ref, bdwg_ref, bdwn_ref,
               bp_ref, out_ref):
    xp = xp_ref[...]
    summed = jnp.dot(xp, bdf_ref[...], preferred_element_type=jnp.float32)
    sq = jnp.dot(xp, bdf2_ref[...], preferred_element_type=jnp.float32)
    nfm_p = 0.5 * (summed * summed - sq)
    gp = (u_ref[0] + u_ref[1]) * (1.0 / (sw_ref[0] + sw_ref[1] + 1e-16))
    out_p = (jnp.dot(gp, bdwg_ref[...], preferred_element_type=jnp.float32)
             + jnp.dot(nfm_p, bdwn_ref[...], preferred_element_type=jnp.float32)
             + bp_ref[...])
    out_ref[...] = out_p


def _post(x_p, bdf, bdf2, u_p, sw_p, bdwg, bdwn, b_p):
    return pl.pallas_call(
        _post_body,
        out_shape=jax.ShapeDtypeStruct((NP, 128), jnp.float32),
    )(x_p, bdf, bdf2, u_p, sw_p, bdwg, bdwn, b_p)


def kernel(x, edge_index, W0, v0, v1, fm_emb, W_proj, b_proj):
    edge_index = edge_index.astype(jnp.int32)
    v01 = jnp.concatenate([v0, v1], axis=1)

    eye8 = jnp.eye(8, dtype=jnp.float32)
    x_p = x.reshape(NP, 8 * D)
    bdw0 = jnp.kron(eye8, W0)
    kv = jnp.kron(eye8, v01)

    h_p, f12i_p = _pre(x_p, bdw0, kv)

    zu = jnp.zeros((N, H), jnp.float32)
    zs = jnp.zeros((N,), jnp.float32)
    u_part, sw = _sc_edges(edge_index, f12i_p.reshape(2 * N),
                           h_p.reshape(N, H), zu, zs)

    bdf = jnp.kron(eye8, fm_emb)
    bdf2 = jnp.kron(eye8, fm_emb * fm_emb)
    bdwg = jnp.kron(eye8, W_proj[:H])
    bdwn = jnp.kron(eye8, W_proj[H:])
    b_p = jnp.tile(b_proj, 8).reshape(1, 128)

    out_p = _post(x_p, bdf, bdf2,
                  u_part.reshape(NC, NP, 128), sw.reshape(NC, NP, 128),
                  bdwg, bdwn, b_p)
    return out_p.reshape(N, OUT)


# speedup vs baseline: 1.3747x; 1.3747x over previous
import functools

import jax
import jax.numpy as jnp
from jax import lax
from jax.experimental import pallas as pl
from jax.experimental.pallas import tpu as pltpu
from jax.experimental.pallas import tpu_sc as plsc

N = 10000
E = 320000
D = 128
H = 16
FM = 16
OUT = 16

NC = 2
NS = 16
NW = NC * NS
EPW = E // NW
G = 80
CH_G = 25
CH = CH_G * G
NCH = EPW // CH
VG = G // 16

NP = N // 8
NSG = N // 16
SG_T = NSG // NS

_SPLAT_DNUMS = lax.GatherDimensionNumbers(
    offset_dims=(), collapsed_slice_dims=(0,), start_index_map=(0,))


def _splat_lane(v, j):
    idx = jnp.full((16, 1), j, jnp.int32)
    return lax.gather(v, idx, _SPLAT_DNUMS, (1,),
                      mode=lax.GatherScatterMode.PROMISE_IN_BOUNDS)


def _pre_body(xp_ref, bdw0_ref, kv_ref, h_ref, f12_ref):
    hb_p = jnp.dot(xp_ref[...], bdw0_ref[...],
                   preferred_element_type=jnp.float32)
    h_ref[...] = hb_p
    f12_ref[...] = jnp.exp(-jnp.dot(hb_p, kv_ref[...],
                                    preferred_element_type=jnp.float32))


def _pre(x_p, bdw0, kv):
    return pl.pallas_call(
        _pre_body,
        out_shape=[
            jax.ShapeDtypeStruct((NP, 128), jnp.float32),
            jax.ShapeDtypeStruct((NP, 16), jnp.float32),
        ],
    )(x_p, bdw0, kv)


def _sc_edges(ei, f12i, h, zu, zs):
    mesh = plsc.VectorSubcoreMesh(core_axis_name="c", subcore_axis_name="s")

    @functools.partial(
        pl.kernel,
        mesh=mesh,
        out_type=[
            jax.ShapeDtypeStruct((NC, N, H), jnp.float32),
            jax.ShapeDtypeStruct((NC, N, 16), jnp.float32),
        ],
        scratch_types=[
            pltpu.VMEM((2 * N,), jnp.float32),
            pltpu.VMEM((EPW,), jnp.int32),
            pltpu.VMEM((EPW,), jnp.int32),
            pltpu.VMEM((CH,), jnp.float32),
            pltpu.VMEM((CH,), jnp.float32),
            pltpu.VMEM((CH, H), jnp.float32),
            pltpu.VMEM((CH, H), jnp.float32),
            pltpu.VMEM(((SG_T + 1) * 16,), jnp.float32),
            pltpu.VMEM_SHARED((N, H), jnp.float32),
            pltpu.VMEM_SHARED((N,), jnp.float32),
            pltpu.SemaphoreType.DMA,
            pltpu.SemaphoreType.DMA,
            pltpu.SemaphoreType.DMA,
            pltpu.SemaphoreType.DMA,
            pltpu.SemaphoreType.DMA,
        ],
        compiler_params=pltpu.CompilerParams(
            needs_layout_passes=False, use_tc_tiling_on_sc=False),
    )
    def k(ei_hbm, f12_hbm, h_hbm, zu_hbm, zs_hbm,
          u_out, sw_out, f12_v, row_f, col_f, val0, val1, hr0, hr1,
          s_loc, u_sh, s_sh, sem_i, sem_g0, sem_g1, sem_s0, sem_s1):
        cid = lax.axis_index("c")
        sid = lax.axis_index("s")
        wid = cid * NS + sid
        base_e = wid * EPW

        vals = (val0, val1)
        hrows = (hr0, hr1)
        sem_g = (sem_g0, sem_g1)
        sem_s = (sem_s0, sem_s1)

        cp_r = pltpu.async_copy(ei_hbm.at[0, pl.ds(base_e, EPW)], row_f, sem_i)
        cp_c = pltpu.async_copy(ei_hbm.at[1, pl.ds(base_e, EPW)], col_f, sem_i)

        @pl.when(sid == 0)
        def _():
            pltpu.sync_copy(zu_hbm, u_sh)
            pltpu.sync_copy(zs_hbm, s_sh)

        pltpu.sync_copy(f12_hbm, f12_v)
        cp_r.wait()
        cp_c.wait()

        plsc.subcore_barrier()

        def fire_gathers(k_ch, b):
            def body(g, _):
                pltpu.async_copy(
                    h_hbm.at[col_f.at[pl.ds(k_ch * CH + g * G, G)]],
                    hrows[b].at[pl.ds(g * G, G)],
                    sem_g[b])
                return 0
            lax.fori_loop(0, CH_G, body, 0)

        def drain_gathers(b):
            pltpu.make_async_copy(h_hbm.at[pl.ds(0, CH)], hrows[b],
                                  sem_g[b]).wait()

        def drain_scatters(b):
            pltpu.make_async_copy(h_hbm.at[pl.ds(0, CH)], hrows[b],
                                  sem_s[b]).wait()
            pltpu.make_async_copy(f12_hbm.at[pl.ds(0, CH)], vals[b],
                                  sem_s[b]).wait()

        def compute_and_scatter(k_ch, b):
            def group_body(g, _):
                def _vg(t, _):
                    loc = g * G + t * 16
                    e0 = k_ch * CH + loc
                    rv = row_f[pl.ds(e0, 16)]
                    cv = col_f[pl.ds(e0, 16)]
                    ag = plsc.load_gather(f12_v, [rv + rv])
                    bg = plsc.load_gather(f12_v, [cv + cv + 1])
                    val = jnp.exp(1.0 / (1.0 + ag * bg))
                    vals[b][pl.ds(loc, 16)] = val
                    for j in range(16):
                        splat = _splat_lane(val, j)
                        hrows[b][loc + j] = hrows[b][loc + j] * splat
                    return 0
                lax.fori_loop(0, VG, _vg, 0)
                idx_slice = row_f.at[pl.ds(k_ch * CH + g * G, G)]
                pltpu.async_copy(hrows[b].at[pl.ds(g * G, G)],
                                 u_sh.at[idx_slice], sem_s[b], add=True)
                pltpu.async_copy(vals[b].at[pl.ds(g * G, G)],
                                 s_sh.at[idx_slice], sem_s[b], add=True)
                return 0
            lax.fori_loop(0, CH_G, group_body, 0)

        fire_gathers(0, 0)
        for k_ch in range(NCH):
            b = k_ch % 2
            nb = 1 - b
            if k_ch >= 1:
                drain_scatters(nb)
            if k_ch + 1 < NCH:
                fire_gathers(k_ch + 1, nb)
            drain_gathers(b)
            compute_and_scatter(k_ch, b)
        drain_scatters((NCH - 1) % 2)

        plsc.subcore_barrier()

        @pl.when(sid == 0)
        def _():
            pltpu.sync_copy(u_sh, u_out.at[cid])

        NPT = SG_T * 16
        pltpu.sync_copy(s_sh.at[pl.ds(sid * NPT, NPT)],
                        s_loc.at[pl.ds(0, NPT)])

        @pl.when(sid == NS - 1)
        def _():
            pltpu.sync_copy(s_sh.at[pl.ds(NS * NPT, 16)],
                            s_loc.at[pl.ds(NPT, 16)])

        def sw_body(i, _):
            s16 = s_loc[pl.ds(i * 16, 16)]
            for j in range(16):
                hr0[i * 16 + j] = _splat_lane(s16, j)
            return 0
        lax.fori_loop(0, SG_T, sw_body, 0)
        pltpu.sync_copy(hr0.at[pl.ds(0, NPT)],
                        sw_out.at[cid, pl.ds(sid * NPT, NPT)])

        @pl.when(sid == NS - 1)
        def _():
            s16 = s_loc[pl.ds(NPT, 16)]
            for j in range(16):
                hr1[j] = _splat_lane(s16, j)
            pltpu.sync_copy(hr1.at[pl.ds(0, 16)],
                            sw_out.at[cid, pl.ds(NS * NPT, 16)])

    return k(ei, f12i, h, zu, zs)


def _post_body(xp_ref, bdf_ref, bdf2_ref, u_ref, sw_ref, bdwg_ref, bdwn_ref,
               bp_ref, out_ref):
    xp = xp_ref[...]
    summed = jnp.dot(xp, bdf_ref[...], preferred_element_type=jnp.float32)
    sq = jnp.dot(xp, bdf2_ref[...], preferred_element_type=jnp.float32)
    nfm_p = 0.5 * (summed * summed - sq)
    gp = (u_ref[0] + u_ref[1]) * (1.0 / (sw_ref[0] + sw_ref[1] + 1e-16))
    out_p = (jnp.dot(gp, bdwg_ref[...], preferred_element_type=jnp.float32)
             + jnp.dot(nfm_p, bdwn_ref[...], preferred_element_type=jnp.float32)
             + bp_ref[...])
    out_ref[...] = out_p


def _post(x_p, bdf, bdf2, u_p, sw_p, bdwg, bdwn, b_p):
    return pl.pallas_call(
        _post_body,
        out_shape=jax.ShapeDtypeStruct((NP, 128), jnp.float32),
    )(x_p, bdf, bdf2, u_p, sw_p, bdwg, bdwn, b_p)


def kernel(x, edge_index, W0, v0, v1, fm_emb, W_proj, b_proj):
    edge_index = edge_index.astype(jnp.int32)
    v01 = jnp.concatenate([v0, v1], axis=1)

    eye8 = jnp.eye(8, dtype=jnp.float32)
    x_p = x.reshape(NP, 8 * D)
    bdw0 = jnp.kron(eye8, W0)
    kv = jnp.kron(eye8, v01)

    h_p, f12i_p = _pre(x_p, bdw0, kv)

    zu = jnp.zeros((N, H), jnp.float32)
    zs = jnp.zeros((N,), jnp.float32)
    u_part, sw = _sc_edges(edge_index, f12i_p.reshape(2 * N),
                           h_p.reshape(N, H), zu, zs)

    bdf = jnp.kron(eye8, fm_emb)
    bdf2 = jnp.kron(eye8, fm_emb * fm_emb)
    bdwg = jnp.kron(eye8, W_proj[:H])
    bdwn = jnp.kron(eye8, W_proj[H:])
    b_p = jnp.tile(b_proj, 8).reshape(1, 128)

    out_p = _post(x_p, bdf, bdf2,
                  u_part.reshape(NC, NP, 128), sw.reshape(NC, NP, 128),
                  bdwg, bdwn, b_p)
    return out_p.reshape(N, OUT)
